# baseline (speedup 1.0000x reference)
import jax
import jax.numpy as jnp
from jax import lax
from jax.experimental import pallas as pl
from jax.experimental.pallas import tpu as pltpu
from jax.experimental.pallas import tpu_sc as plsc

BATCH = 16384
F = 32
L = 1_000_000
LS = 1000

_NC = 2
_NS = 16
_NW = _NC * _NS
_BPW = BATCH // _NW

_CHUNK = 1024
_RPW = 30
_LPW = 31232
_TAIL0 = 999936
_NGRP = BATCH // 16
_CRING = 2
_RING = 6
_BSH = 14

_SROWS = BATCH + 16


def _body_a(pc_idx_hbm, wp_t_hbm, tail_p_hbm, stage_hbm,
            idxv, plist, chunkring, epi, rbring, dixring, fsem, esem, ssem):
    wid = lax.axis_index("s") * _NC + lax.axis_index("c")
    lo = wid * _LPW
    hi = jnp.where(wid == _NW - 1, jnp.int32(L), lo + _LPW)
    lane16 = lax.iota(jnp.int32, 16)

    def fetch(k):
        fs = pl.multiple_of(lo + k * _CHUNK, 128)
        pltpu.async_copy(wp_t_hbm.at[:, pl.ds(fs, _CHUNK)],
                         chunkring.at[lax.rem(k, _CRING)], fsem)

    fetch(0)
    pltpu.async_copy(
        wp_t_hbm.at[:, pl.ds(pl.multiple_of(lo + _RPW * _CHUNK, 128), 512)],
        epi, esem)

    pltpu.sync_copy(pc_idx_hbm, idxv.at[pl.ds(0, BATCH)])

    def scan_body(i, cursor):
        j16 = idxv[pl.ds(i * 16, 16)]
        b16 = lane16 + i * 16
        m = (j16 >= lo) & (j16 < hi)
        n = plsc.all_reduce_population_count(m)[0]
        p16 = ((j16 - lo) << _BSH) | b16
        plsc.store_compressed(plist.at[pl.ds(cursor, 16)], p16, mask=m)
        return cursor + n

    count = lax.fori_loop(0, _NGRP, scan_body, jnp.int32(0), unroll=4)
    plist[pl.ds(count, 16)] = jnp.full((16,), -1, jnp.int32)
    ngroups = (count + 15) // 16

    def bin_pass(q, cur):
        def bb(g, c):
            p16 = plist[pl.ds(g * 16, 16)]
            m = (p16 >> (_BSH + 13)) == q
            n = plsc.all_reduce_population_count(m)[0]
            plsc.store_compressed(idxv.at[pl.ds(c, 16)], p16, mask=m)
            return c + n
        return lax.fori_loop(0, ngroups, bb, cur)

    qoff = [jnp.int32(0)]
    for q in range(4):
        qoff.append(bin_pass(q, qoff[-1]))
    idxv[pl.ds(qoff[-1], 16)] = jnp.full((16,), -1, jnp.int32)

    def qbounds(qt):
        s = jnp.where(qt == 0, qoff[0],
                      jnp.where(qt == 1, qoff[1],
                                jnp.where(qt == 2, qoff[2], qoff[3])))
        e = jnp.where(qt == 0, qoff[1],
                      jnp.where(qt == 1, qoff[2],
                                jnp.where(qt == 2, qoff[3], qoff[4])))
        return s // 16, (e + 15) // 16

    def process_chunk(chunk, rlo, rhi, fired0, gs, ge, lmax=_CHUNK - 1):
        plo = rlo << _BSH
        phi = rhi << _BSH

        def comp(g, cur):
            p16 = idxv[pl.ds(g * 16, 16)]
            m = (p16 >= plo) & (p16 < phi)
            n = plsc.all_reduce_population_count(m)[0]
            plsc.store_compressed(plist.at[pl.ds(cur, 16)], p16, mask=m)
            return cur + n

        cnt = lax.fori_loop(gs, ge, comp, jnp.int32(0))

        def ext(e, fired):
            base = e * 16
            p16 = plist[pl.ds(base, 16)]
            m = lane16 < (cnt - base)
            b16 = p16 & ((1 << _BSH) - 1)
            slot = lax.rem(fired, _RING)

            @pl.when(fired >= _RING)
            def _():
                pltpu.make_async_copy(stage_hbm.at[pl.ds(0, 16)],
                                      rbring.at[0], ssem).wait()

            rb = rbring.at[slot]
            dix = dixring.at[slot]
            for ee in range(16):
                p = p16[ee]
                l = lax.max(jnp.int32(0),
                            lax.min((p >> _BSH) - rlo, jnp.int32(lmax)))
                lb = jnp.broadcast_to(l, (16,))
                rb[ee, pl.ds(0, 16)] = plsc.load_gather(chunk, [lane16, lb])
                rb[ee, pl.ds(16, 16)] = plsc.load_gather(
                    chunk, [lane16 + 16, lb])
            dix[0, :] = jnp.where(m, b16, _SROWS - 16 + lane16)
            pltpu.async_copy(rb, stage_hbm.at[dix.at[0]], ssem)
            return fired + 1

        return lax.fori_loop(0, (cnt + 15) // 16, ext, fired0)

    def chunk_body(k, fired):
        @pl.when(k + (_CRING - 1) < _RPW)
        def _():
            fetch(k + (_CRING - 1))

        pltpu.make_async_copy(wp_t_hbm.at[:, pl.ds(0, _CHUNK)],
                              chunkring.at[0], fsem).wait()
        rlo = k * _CHUNK
        gs, ge = qbounds(k >> 3)
        return process_chunk(chunkring.at[lax.rem(k, _CRING)], rlo,
                             rlo + _CHUNK, fired, gs, ge)

    fired = lax.fori_loop(0, _RPW, chunk_body, jnp.int32(0))

    gs3 = qoff[3] // 16
    ge3 = (qoff[4] + 15) // 16

    pltpu.make_async_copy(wp_t_hbm.at[:, pl.ds(0, 512)], epi, esem).wait()
    fired = process_chunk(epi, jnp.int32(_RPW * _CHUNK),
                          jnp.int32(_LPW), fired, gs3, ge3, lmax=511)

    def do_extra(f0):
        pltpu.sync_copy(wp_t_hbm.at[:, pl.ds(999424, 512)], epi)
        f1 = process_chunk(epi, jnp.int32(_LPW), jnp.int32(_LPW + 512), f0,
                           gs3, ge3, lmax=511)
        pltpu.sync_copy(tail_p_hbm, epi.at[:, pl.ds(0, 128)])
        return process_chunk(epi, jnp.int32(_TAIL0 - 968192),
                             jnp.int32(L - 968192), f1, gs3, ge3, lmax=127)

    fired = lax.cond(wid == _NW - 1, do_extra, lambda f: f, fired)

    def drain(i, _):
        pltpu.make_async_copy(stage_hbm.at[pl.ds(0, 16)],
                              rbring.at[0], ssem).wait()
        return 0

    lax.fori_loop(0, jnp.minimum(fired, _RING), drain, 0)


def _body_b(stage_hbm, stars_idx_hbm, ws_t_hbm, out_hbm,
            sidx, sbuf, stvring, outblock, fsem, sem):
    wid = lax.axis_index("s") * _NC + lax.axis_index("c")
    b0 = wid * _BPW
    quarter = _BPW // 4
    nq = 4

    c1 = pltpu.async_copy(stars_idx_hbm.at[pl.ds(b0, _BPW)], sidx, sem)
    c2 = pltpu.async_copy(ws_t_hbm, sbuf, sem)
    pltpu.async_copy(stage_hbm.at[pl.ds(b0, quarter)], stvring.at[0], fsem)
    c1.wait()
    c2.wait()

    for q in range(nq):
        if q + 1 < nq:
            pltpu.async_copy(
                stage_hbm.at[pl.ds(b0 + (q + 1) * quarter, quarter)],
                stvring.at[(q + 1) % 2], fsem)
        pltpu.make_async_copy(stage_hbm.at[pl.ds(0, quarter)],
                              stvring.at[0], fsem).wait()
        stv = stvring.at[q % 2]

        def transpose_group(g, _, q=q, stv=stv):
            gg = g + q * (quarter // 16)
            b16l = lax.iota(jnp.int32, 16) + g * 16
            j16 = sidx[pl.ds(gg * 16, 16)]
            for f in range(F):
                svals = plsc.load_gather(
                    sbuf, [jnp.full((16,), f, jnp.int32), j16])
                outblock[f, pl.ds(gg * 16, 16)] = svals
                pvals = plsc.load_gather(
                    stv, [b16l, jnp.full((16,), f, jnp.int32)])
                outblock[F + f, pl.ds(gg * 16, 16)] = pvals
            return 0

        lax.fori_loop(0, quarter // 16, transpose_group, 0)

    pltpu.sync_copy(outblock, out_hbm.at[:, pl.ds(b0, _BPW)])


@jax.jit
def _run(stars_idx, postalcode_idx, W_stars, W_postalcode):
    mesh = plsc.VectorSubcoreMesh(core_axis_name="c", subcore_axis_name="s")
    params = pltpu.CompilerParams(needs_layout_passes=False)

    ka = pl.kernel(
        _body_a,
        out_type=jax.ShapeDtypeStruct((_SROWS, 128), jnp.float32),
        mesh=mesh,
        scratch_types=[
            pltpu.VMEM((BATCH + 16,), jnp.int32),
            pltpu.VMEM((BATCH + 16,), jnp.int32),
            pltpu.VMEM((_CRING, F, _CHUNK), jnp.float32),
            pltpu.VMEM((F, 512), jnp.float32),
            pltpu.VMEM((_RING, 16, 128), jnp.float32),
            pltpu.VMEM((_RING, 1, 16), jnp.int32),
            pltpu.SemaphoreType.DMA,
            pltpu.SemaphoreType.DMA,
            pltpu.SemaphoreType.DMA,
        ],
        compiler_params=params,
    )
    kb = pl.kernel(
        _body_b,
        out_type=jax.ShapeDtypeStruct((2 * F, BATCH), jnp.float32),
        mesh=mesh,
        scratch_types=[
            pltpu.VMEM((_BPW,), jnp.int32),
            pltpu.VMEM((F, LS), jnp.float32),
            pltpu.VMEM((2, _BPW // 4, 128), jnp.float32),
            pltpu.VMEM((2 * F, _BPW), jnp.float32),
            pltpu.SemaphoreType.DMA,
            pltpu.SemaphoreType.DMA,
        ],
        compiler_params=params,
    )

    pc_idx = postalcode_idx.astype(jnp.int32)
    s_idx = stars_idx.astype(jnp.int32)
    wp_t = W_postalcode.T
    ws_t = W_stars.T
    tail_p = jnp.pad(W_postalcode[_TAIL0:].T, ((0, 0), (0, 128 - (L - _TAIL0))))

    stage = ka(pc_idx, wp_t, tail_p)
    out_t = kb(stage, s_idx, ws_t)
    return out_t.T


def kernel(stars_idx, postalcode_idx, W_stars, W_postalcode):
    return _run(stars_idx, postalcode_idx, W_stars, W_postalcode)
